# baseline (speedup 1.0000x reference)
import functools

import jax
import jax.numpy as jnp
from jax import lax
from jax.experimental import pallas as pl
from jax.experimental.pallas import tpu as pltpu
from jax.experimental.pallas import tpu_sc as plsc

_MARGIN = 0.3
_SCALE = 32.0
_B = 4096
_C = 10000
_NC, _NS, _L = 2, 16, 16
_NW = _NC * _NS
_NSL = 4
_BS = _B // _NSL
_NPW = _BS // _NW
_RPC = 4
_NCHUNK = _NPW // _RPC
_U = 8
_NFULL = (_C // _L - 1) // _U


def _sc_loss_body(cos_hbm, lbl_hbm, s_hbm, g_hbm, buf0, buf1, lbl_v, res_v,
                  g_v, sem0, sem1, lsem):
    wid = lax.axis_index("s") * _NC + lax.axis_index("c")
    row0 = wid * _NPW

    pltpu.sync_copy(lbl_hbm.at[pl.ds(row0, _NPW)], lbl_v)
    pltpu.async_copy(cos_hbm.at[pl.ds(row0, _RPC)], buf0, sem0)
    pltpu.async_copy(cos_hbm.at[pl.ds(row0 + _RPC, _RPC)], buf1, sem1)

    zero16 = jnp.zeros((_L,), jnp.float32)
    iota16 = lax.iota(jnp.int32, _L)

    def do_chunk(g, buf):
        for rr in range(_RPC):
            loc = g * _RPC + rr
            loc16 = iota16 * 0 + loc
            lbl16 = plsc.load_gather(lbl_v, [loc16])
            rr16 = iota16 * 0 + rr
            craw16 = plsc.load_gather(buf, [rr16, lbl16])
            thr16 = craw16 - _MARGIN

            def inner(j, accs):
                a0, a1 = accs
                off = j * (_U * _L)
                for u in range(_U):
                    v = buf[rr, pl.ds(off + u * _L, _L)]
                    x = jnp.where(v >= thr16, v * _SCALE - _SCALE, -_SCALE)
                    e = jnp.exp(x)
                    if u % 2 == 0:
                        a0 = a0 + e
                    else:
                        a1 = a1 + e
                return (a0, a1)

            a0, a1 = lax.fori_loop(0, _NFULL, inner, (zero16, zero16))
            v = buf[rr, pl.ds(_C - _L, _L)]
            x = jnp.where(v >= thr16, v * _SCALE - _SCALE, -_SCALE)
            acc = a0 + a1 + jnp.exp(x)
            plsc.store_scatter(res_v, [loc * _L + iota16], acc)
            plsc.store_scatter(g_v, [loc * _L + iota16], craw16)

    def ring(i, carry):
        g = i * 2
        pltpu.make_async_copy(cos_hbm.at[pl.ds(row0, _RPC)], buf0, sem0).wait()
        do_chunk(g, buf0)

        @pl.when(g + 2 < _NCHUNK)
        def _():
            pltpu.async_copy(
                cos_hbm.at[pl.ds(row0 + (g + 2) * _RPC, _RPC)], buf0, sem0
            )

        pltpu.make_async_copy(cos_hbm.at[pl.ds(row0, _RPC)], buf1, sem1).wait()
        do_chunk(g + 1, buf1)

        @pl.when(g + 3 < _NCHUNK)
        def _():
            pltpu.async_copy(
                cos_hbm.at[pl.ds(row0 + (g + 3) * _RPC, _RPC)], buf1, sem1
            )

        return carry

    lax.fori_loop(0, _NCHUNK // 2, ring, 0)

    pltpu.sync_copy(res_v, s_hbm.at[pl.ds(row0 * _L, _NPW * _L)])
    pltpu.sync_copy(g_v, g_hbm.at[pl.ds(row0 * _L, _NPW * _L)])
    del lsem


@functools.partial(
    pl.kernel,
    out_type=(
        jax.ShapeDtypeStruct((_BS * _L,), jnp.float32),
        jax.ShapeDtypeStruct((_BS * _L,), jnp.float32),
    ),
    mesh=plsc.VectorSubcoreMesh(core_axis_name="c", subcore_axis_name="s"),
    scratch_types=[
        pltpu.VMEM((_RPC, _C), jnp.float32),
        pltpu.VMEM((_RPC, _C), jnp.float32),
        pltpu.VMEM((_NPW,), jnp.int32),
        pltpu.VMEM((_NPW * _L,), jnp.float32),
        pltpu.VMEM((_NPW * _L,), jnp.float32),
        pltpu.SemaphoreType.DMA,
        pltpu.SemaphoreType.DMA,
        pltpu.SemaphoreType.DMA,
    ],
    compiler_params=pltpu.CompilerParams(needs_layout_passes=False),
)
def _sc_loss(cos_hbm, lbl_hbm, s_hbm, g_hbm, buf0, buf1, lbl_v, res_v, g_v,
             sem0, sem1, lsem):
    _sc_loss_body(cos_hbm, lbl_hbm, s_hbm, g_hbm, buf0, buf1, lbl_v, res_v,
                  g_v, sem0, sem1, lsem)


def _finish_body(s_ref, g_ref, out_ref):
    s_sum = jnp.sum(s_ref[...], axis=1, keepdims=True)
    craw = g_ref[:, 0:1]
    t = _SCALE * (craw - _MARGIN)
    corr = jnp.exp(_SCALE * craw - _SCALE)
    s_corr = s_sum - corr + jnp.exp(t - _SCALE)
    lse = _SCALE + jnp.log(s_corr)
    out_ref[...] = jnp.sum(lse - t, keepdims=True)


def _finish(s_parts, g_parts):
    return pl.pallas_call(
        _finish_body,
        grid=(1,),
        in_specs=[
            pl.BlockSpec((_B, _L), lambda i: (0, 0)),
            pl.BlockSpec((_B, _L), lambda i: (0, 0)),
        ],
        out_specs=pl.BlockSpec((1, 1), lambda i: (0, 0)),
        out_shape=jax.ShapeDtypeStruct((1, 1), jnp.float32),
    )(s_parts, g_parts)


def kernel(cosine, label):
    b, c = cosine.shape
    s_parts, g_parts = [], []
    for si in range(_NSL):
        cs = lax.slice(cosine, (si * _BS, 0), ((si + 1) * _BS, c))
        ls = lax.slice(label, (si * _BS,), ((si + 1) * _BS,))
        s_flat, g_flat = _sc_loss(cs, ls)
        s_parts.append(s_flat.reshape(_BS, _L))
        g_parts.append(g_flat.reshape(_BS, _L))
    out = _finish(jnp.concatenate(s_parts, axis=0),
                  jnp.concatenate(g_parts, axis=0))
    return (out[0, 0] / b).reshape(())
